# Initial kernel scaffold; baseline (speedup 1.0000x reference)
import jax
import jax.numpy as jnp
from jax.experimental import pallas as pl


def kernel(hidden_states, group_ids, ln_summary_w, ln_cross_w, Wq, Wk, Wv, Wo, W_gate, b_gate, W_broadcast):
    raise NotImplementedError("write your pallas kernel here")


# speedup vs baseline: 1.3952x; 1.3952x over previous
import functools

import jax
import jax.numpy as jnp
from jax import lax
from jax.experimental import pallas as pl
from jax.experimental.pallas import tpu as pltpu

B = 64
S = 512
D = 1024
G = 16
TOP_K = 4
EPS = 1e-06
SCALE = D ** -0.5
NEG = float(jnp.finfo(jnp.float32).min)


def _dot_t(a, b):
    return lax.dot_general(a, b, (((1,), (1,)), ((), ())),
                           preferred_element_type=jnp.float32)


def _summarize_body(h_ref, w_ref, out_ref):
    h = h_ref[0]
    var = jnp.mean(h * h, axis=-1, keepdims=True)
    normed = w_ref[...] * (h * lax.rsqrt(var + EPS))
    out_ref[0] = jnp.mean(normed, axis=0, keepdims=True)


def _attention_body(ps_ref, gid_ref, lnw_ref, wq_ref, wk_ref, wv_ref,
                    wo_ref, wb_ref, wgc_ref, bg_ref,
                    attn_ref, ct_ref, zc_ref):
    per_series = ps_ref[...]
    gids = gid_ref[...]
    rows = lax.broadcasted_iota(jnp.int32, (G, B), 0)
    onehot = (rows == gids).astype(jnp.float32)
    sums = jnp.dot(onehot, per_series, preferred_element_type=jnp.float32)
    counts = jnp.sum(onehot, axis=1, keepdims=True)
    summaries = sums / jnp.maximum(counts, 1.0)

    var = jnp.mean(summaries * summaries, axis=-1, keepdims=True)
    normed = lnw_ref[...] * (summaries * lax.rsqrt(var + EPS))

    q = _dot_t(normed, wq_ref[...])
    k = _dot_t(normed, wk_ref[...])
    v = _dot_t(normed, wv_ref[...])
    scores = _dot_t(q, k) * SCALE

    col = lax.broadcasted_iota(jnp.int32, (G, G), 1)
    mask = lax.broadcasted_iota(jnp.int32, (G, G), 0) == col
    work = scores
    for _ in range(TOP_K):
        m = jnp.max(work, axis=1, keepdims=True)
        is_max = work == m
        first = jnp.min(jnp.where(is_max, col, G), axis=1, keepdims=True)
        sel = col == first
        mask = mask | sel
        work = jnp.where(sel, NEG, work)

    masked = jnp.where(mask, scores, NEG)
    mx = jnp.max(masked, axis=1, keepdims=True)
    e = jnp.exp(masked - mx)
    attn = e / jnp.sum(e, axis=1, keepdims=True)
    attn_ref[...] = attn

    cross = jnp.dot(attn, v, preferred_element_type=jnp.float32)
    cross = _dot_t(cross, wo_ref[...])
    ct = _dot_t(cross, wb_ref[...])
    zc = _dot_t(ct, wgc_ref[...]) + bg_ref[...]
    ct_ref[...] = ct
    zc_ref[...] = zc


def _fuse_body(gid_ref, h_ref, wgh_ref, zc_ref, ct_ref, out_ref):
    h = h_ref[0]
    z = _dot_t(h, wgh_ref[...]) + zc_ref[0]
    gate = 1.0 / (1.0 + jnp.exp(-z))
    out_ref[0] = h + gate * ct_ref[0]


def kernel(hidden_states, group_ids, ln_summary_w, ln_cross_w, Wq, Wk, Wv,
           Wo, W_gate, b_gate, W_broadcast):
    gids = group_ids.astype(jnp.int32)
    lnw_s = ln_summary_w.reshape(1, D)
    lnw_c = ln_cross_w.reshape(1, D)
    wg_h = W_gate[:, :D]
    wg_c = W_gate[:, D:]
    bg = b_gate.reshape(1, D)

    per_series = pl.pallas_call(
        _summarize_body,
        grid=(B,),
        in_specs=[
            pl.BlockSpec((1, S, D), lambda b: (b, 0, 0)),
            pl.BlockSpec((1, D), lambda b: (0, 0)),
        ],
        out_specs=pl.BlockSpec((1, 1, D), lambda b: (b, 0, 0)),
        out_shape=jax.ShapeDtypeStruct((B, 1, D), jnp.float32),
    )(hidden_states, lnw_s)
    per_series = per_series.reshape(B, D)

    attn, ct_g, zc_g = pl.pallas_call(
        _attention_body,
        out_shape=(
            jax.ShapeDtypeStruct((G, G), jnp.float32),
            jax.ShapeDtypeStruct((G, D), jnp.float32),
            jax.ShapeDtypeStruct((G, D), jnp.float32),
        ),
    )(per_series, gids.reshape(1, B), lnw_c, Wq, Wk, Wv, Wo,
      W_broadcast, wg_c, bg)
    ct_g = ct_g.reshape(G, 1, D)
    zc_g = zc_g.reshape(G, 1, D)

    out = pl.pallas_call(
        _fuse_body,
        grid_spec=pltpu.PrefetchScalarGridSpec(
            num_scalar_prefetch=1,
            grid=(B,),
            in_specs=[
                pl.BlockSpec((1, S, D), lambda b, g: (b, 0, 0)),
                pl.BlockSpec((D, D), lambda b, g: (0, 0)),
                pl.BlockSpec((1, 1, D), lambda b, g: (g[b], 0, 0)),
                pl.BlockSpec((1, 1, D), lambda b, g: (g[b], 0, 0)),
            ],
            out_specs=pl.BlockSpec((1, S, D), lambda b, g: (b, 0, 0)),
        ),
        out_shape=jax.ShapeDtypeStruct((B, S, D), jnp.float32),
    )(gids, hidden_states, wg_h, zc_g, ct_g)

    return (out, attn)
